# Initial kernel scaffold; baseline (speedup 1.0000x reference)
import jax
import jax.numpy as jnp
from jax.experimental import pallas as pl


def kernel(idx, targets, table):
    raise NotImplementedError("write your pallas kernel here")


# speedup vs baseline: 1.5223x; 1.5223x over previous
import functools

import jax
import jax.numpy as jnp
from jax.experimental import pallas as pl
from jax.experimental.pallas import tpu as pltpu


def _round_up(x, m):
    return (x + m - 1) // m * m


def _row_lse_kernel(table_ref, lse_ref):
    t = table_ref[...]
    m = jnp.max(t, axis=1, keepdims=True)
    lse = jnp.log(jnp.sum(jnp.exp(t - m), axis=1, keepdims=True)) + m
    lse_ref[...] = lse.reshape(1, -1)


def _fwd_loss_kernel(idx_ref, tgt_ref, table_ref, lse_ref,
                     logits_ref, rowloss_ref):
    tn, vpad = logits_ref.shape
    idx = idx_ref[...]
    tgt = tgt_ref[...]

    iota = jax.lax.broadcasted_iota(jnp.int32, (tn, vpad), 1)
    sel = idx == iota
    one_hot = jnp.where(sel, 1.0, 0.0).astype(jnp.bfloat16)
    logits = jnp.dot(one_hot, table_ref[...],
                     preferred_element_type=jnp.float32)
    logits_ref[...] = logits

    lse_n = jnp.sum(jnp.where(sel, lse_ref[...], 0.0),
                    axis=1, keepdims=True)
    correct = jnp.sum(jnp.where(tgt == iota, logits, 0.0),
                      axis=1, keepdims=True)
    rowloss_ref[...] = lse_n - correct


def _fwd_logits_kernel(idx_ref, table_ref, logits_ref):
    tn, vpad = logits_ref.shape
    iota = jax.lax.broadcasted_iota(jnp.int32, (tn, vpad), 1)
    one_hot = jnp.where(idx_ref[...] == iota, 1.0, 0.0).astype(jnp.bfloat16)
    logits_ref[...] = jnp.dot(one_hot, table_ref[...],
                              preferred_element_type=jnp.float32)


@functools.partial(jax.jit, static_argnames=("tn_max",))
def _forward(idx, targets, table, *, tn_max=1024):
    B, T = idx.shape
    V = table.shape[0]
    N = B * T

    Vpad = _round_up(V, 128)
    TN = min(_round_up(tn_max, 8), _round_up(N, 8))
    Npad = _round_up(N, TN)
    num_tiles = Npad // TN
    has_targets = targets is not None

    table_f32 = table.astype(jnp.float32)
    table_pad = jnp.pad(table_f32, ((0, Vpad - V), (0, Vpad - V)))
    if has_targets and Vpad > V:
        table_pad = table_pad.at[:, V:].set(jnp.float32(-1e30))
    table_bf16 = table_pad.astype(jnp.bfloat16)

    idx_flat = jnp.pad(idx.reshape(-1).astype(jnp.int32),
                       (0, Npad - N)).reshape(Npad, 1)

    vmem_limit = int(min(96 * 1024 * 1024,
                         max(8 * 1024 * 1024,
                             8 * TN * Vpad * 4 + 4 * Vpad * Vpad * 4)))
    cparams = pltpu.CompilerParams(
        dimension_semantics=("parallel",), vmem_limit_bytes=vmem_limit)

    row_spec = pl.BlockSpec((TN, 1), lambda i: (i, 0))
    table_spec = pl.BlockSpec((Vpad, Vpad), lambda i: (0, 0))
    lse_spec = pl.BlockSpec((1, Vpad), lambda i: (0, 0))
    logits_spec = pl.BlockSpec((TN, Vpad), lambda i: (i, 0))

    if has_targets:
        row_lse = pl.pallas_call(
            _row_lse_kernel,
            out_shape=jax.ShapeDtypeStruct((1, Vpad), jnp.float32),
        )(table_pad)

        tgt_flat = jnp.pad(targets.reshape(-1).astype(jnp.int32),
                           (0, Npad - N)).reshape(Npad, 1)
        logits_pad, rowloss = pl.pallas_call(
            _fwd_loss_kernel,
            out_shape=(
                jax.ShapeDtypeStruct((Npad, Vpad), jnp.float32),
                jax.ShapeDtypeStruct((Npad, 1), jnp.float32),
            ),
            grid_spec=pltpu.PrefetchScalarGridSpec(
                num_scalar_prefetch=0,
                grid=(num_tiles,),
                in_specs=[row_spec, row_spec, table_spec, lse_spec],
                out_specs=[logits_spec, row_spec],
            ),
            compiler_params=cparams,
        )(idx_flat, tgt_flat, table_bf16, row_lse)
        logits_flat = logits_pad[:N, :V]
        loss = jnp.sum(rowloss[:N, 0]) / jnp.float32(N)
        return logits_flat, loss

    logits_pad = pl.pallas_call(
        _fwd_logits_kernel,
        out_shape=jax.ShapeDtypeStruct((Npad, Vpad), jnp.float32),
        grid_spec=pltpu.PrefetchScalarGridSpec(
            num_scalar_prefetch=0,
            grid=(num_tiles,),
            in_specs=[row_spec, table_spec],
            out_specs=logits_spec,
        ),
        compiler_params=cparams,
    )(idx_flat, table_bf16)
    return logits_pad[:N, :V].reshape(B, T, V), None


def kernel(idx, targets, table):
    return _forward(idx, targets, table)


# speedup vs baseline: 1.5224x; 1.0001x over previous
import functools

import jax
import jax.numpy as jnp
from jax.experimental import pallas as pl
from jax.experimental.pallas import tpu as pltpu


def _round_up(x, m):
    return (x + m - 1) // m * m


def _row_lse_kernel(table_ref, lse_ref):
    t = table_ref[...]
    m = jnp.max(t, axis=1, keepdims=True)
    lse = jnp.log(jnp.sum(jnp.exp(t - m), axis=1, keepdims=True)) + m
    lse_ref[...] = lse.reshape(1, -1)


def _fwd_loss_kernel(idx_ref, tgt_ref, table_ref, lse_ref,
                     logits_ref, rowloss_ref):
    tn, vpad = logits_ref.shape
    idx = idx_ref[...]
    tgt = tgt_ref[...]

    iota = jax.lax.broadcasted_iota(jnp.int32, (tn, vpad), 1)
    sel = idx == iota
    one_hot = jnp.where(sel, 1.0, 0.0).astype(jnp.bfloat16)
    logits = jnp.dot(one_hot, table_ref[...],
                     preferred_element_type=jnp.float32)
    logits_ref[...] = logits

    lse_n = jnp.sum(jnp.where(sel, lse_ref[...], 0.0),
                    axis=1, keepdims=True)
    correct = jnp.sum(jnp.where(tgt == iota, logits, 0.0),
                      axis=1, keepdims=True)
    rowloss_ref[...] = lse_n - correct


def _fwd_logits_kernel(idx_ref, table_ref, logits_ref):
    tn, vpad = logits_ref.shape
    iota = jax.lax.broadcasted_iota(jnp.int32, (tn, vpad), 1)
    one_hot = jnp.where(idx_ref[...] == iota, 1.0, 0.0).astype(jnp.bfloat16)
    logits_ref[...] = jnp.dot(one_hot, table_ref[...],
                              preferred_element_type=jnp.float32)


@functools.partial(jax.jit, static_argnames=("tn_max",))
def _forward(idx, targets, table, *, tn_max=1024):
    B, T = idx.shape
    V = table.shape[0]
    N = B * T

    Vpad = _round_up(V, 128)
    TN = min(_round_up(tn_max, 8), _round_up(N, 8))
    Npad = _round_up(N, TN)
    num_tiles = Npad // TN
    has_targets = targets is not None

    table_f32 = table.astype(jnp.float32)
    table_pad = jnp.pad(table_f32, ((0, Vpad - V), (0, Vpad - V)))
    if has_targets and Vpad > V:
        table_pad = table_pad.at[:, V:].set(jnp.float32(-1e30))
    table_bf16 = table_pad.astype(jnp.bfloat16)

    idx_flat = idx.reshape(-1).astype(jnp.int32)
    if Npad > N:
        idx_flat = jnp.pad(idx_flat, (0, Npad - N))
    idx_flat = idx_flat.reshape(Npad, 1)

    vmem_limit = int(min(96 * 1024 * 1024,
                         max(8 * 1024 * 1024,
                             8 * TN * Vpad * 4 + 4 * Vpad * Vpad * 4)))
    cparams = pltpu.CompilerParams(
        dimension_semantics=("parallel",), vmem_limit_bytes=vmem_limit)

    row_spec = pl.BlockSpec((TN, 1), lambda i: (i, 0))
    table_spec = pl.BlockSpec((Vpad, Vpad), lambda i: (0, 0))
    lse_spec = pl.BlockSpec((1, Vpad), lambda i: (0, 0))
    logits_spec = pl.BlockSpec((TN, Vpad), lambda i: (i, 0))

    if has_targets:
        row_lse = pl.pallas_call(
            _row_lse_kernel,
            out_shape=jax.ShapeDtypeStruct((1, Vpad), jnp.float32),
        )(table_pad)

        tgt_flat = targets.reshape(-1).astype(jnp.int32)
        if Npad > N:
            tgt_flat = jnp.pad(tgt_flat, (0, Npad - N))
        tgt_flat = tgt_flat.reshape(Npad, 1)
        logits_pad, rowloss = pl.pallas_call(
            _fwd_loss_kernel,
            out_shape=(
                jax.ShapeDtypeStruct((Npad, Vpad), jnp.float32),
                jax.ShapeDtypeStruct((Npad, 1), jnp.float32),
            ),
            grid_spec=pltpu.PrefetchScalarGridSpec(
                num_scalar_prefetch=0,
                grid=(num_tiles,),
                in_specs=[row_spec, row_spec, table_spec, lse_spec],
                out_specs=[logits_spec, row_spec],
            ),
            compiler_params=cparams,
        )(idx_flat, tgt_flat, table_bf16, row_lse)
        if Npad > N or Vpad > V:
            logits_flat = logits_pad[:N, :V]
            rowloss = rowloss[:N]
        else:
            logits_flat = logits_pad
        loss = jnp.sum(rowloss[:, 0]) / jnp.float32(N)
        return logits_flat, loss

    logits_pad = pl.pallas_call(
        _fwd_logits_kernel,
        out_shape=jax.ShapeDtypeStruct((Npad, Vpad), jnp.float32),
        grid_spec=pltpu.PrefetchScalarGridSpec(
            num_scalar_prefetch=0,
            grid=(num_tiles,),
            in_specs=[row_spec, table_spec],
            out_specs=logits_spec,
        ),
        compiler_params=cparams,
    )(idx_flat, table_bf16)
    if Npad > N or Vpad > V:
        logits_pad = logits_pad[:N, :V]
    return logits_pad.reshape(B, T, V), None


def kernel(idx, targets, table):
    return _forward(idx, targets, table)


# speedup vs baseline: 4.6409x; 3.0485x over previous
import functools

import jax
import jax.numpy as jnp
from jax.experimental import pallas as pl
from jax.experimental.pallas import tpu as pltpu


def _round_up(x, m):
    return (x + m - 1) // m * m


def _lse_m_kernel(table_ref, m_ref, lse_ref):
    t = table_ref[...]
    mx = jnp.max(t, axis=1, keepdims=True)
    lse = jnp.log(jnp.sum(jnp.exp(t - mx), axis=1, keepdims=True)) + mx
    m_ref[...] = lse - t
    lse_ref[...] = lse.reshape(1, -1)


def _loss_tile_kernel(idx_ref, tgt_ref, table_ref, m_ref,
                      logits_ref, partial_ref):
    tblk, vpad = logits_ref.shape
    tok = idx_ref[...].reshape(1, tblk)
    tgt = tgt_ref[...].reshape(1, tblk)

    viota = jax.lax.broadcasted_iota(jnp.int32, (vpad, tblk), 0)
    p = jnp.where(tok == viota, 1.0, 0.0).astype(jnp.bfloat16)
    q = jnp.where(tgt == viota, 1.0, 0.0).astype(jnp.bfloat16)

    logits_ref[...] = jax.lax.dot_general(
        p, table_ref[...], (((0,), (0,)), ((), ())),
        preferred_element_type=jnp.float32)

    c = jax.lax.dot_general(
        p, q, (((1,), (1,)), ((), ())),
        preferred_element_type=jnp.float32)
    partial = jnp.sum(c * m_ref[...], axis=0, keepdims=True)
    partial_ref[...] = partial.reshape(1, 1, vpad)


def _logits_tile_kernel(idx_ref, table_ref, logits_ref):
    tblk, vpad = logits_ref.shape
    tok = idx_ref[...].reshape(1, tblk)
    viota = jax.lax.broadcasted_iota(jnp.int32, (vpad, tblk), 0)
    p = jnp.where(tok == viota, 1.0, 0.0).astype(jnp.bfloat16)
    logits_ref[...] = jax.lax.dot_general(
        p, table_ref[...], (((0,), (0,)), ((), ())),
        preferred_element_type=jnp.float32)


@functools.partial(jax.jit, static_argnames=("tblk",))
def _forward(idx, targets, table, *, tblk=512):
    B, T = idx.shape
    V = table.shape[0]
    N = B * T

    Vpad = _round_up(V, 128)
    TBLK = min(tblk, _round_up(N, 128))
    Npad = _round_up(N, TBLK)
    num_tiles = Npad // TBLK
    has_targets = targets is not None

    table_f32 = table.astype(jnp.float32)
    table_pad = jnp.pad(table_f32, ((0, Vpad - V), (0, Vpad - V)))
    if has_targets and Vpad > V:
        table_pad = table_pad.at[:, V:].set(jnp.float32(-1e30))
    table_bf16 = table_pad.astype(jnp.bfloat16)

    idx_flat = idx.reshape(-1).astype(jnp.int32)
    if Npad > N:
        idx_flat = jnp.pad(idx_flat, (0, Npad - N))
    idx3 = idx_flat.reshape(num_tiles, 1, TBLK)

    vmem_limit = int(min(96 * 1024 * 1024,
                         max(8 * 1024 * 1024,
                             8 * TBLK * Vpad * 4 + 6 * Vpad * Vpad * 4)))
    cparams = pltpu.CompilerParams(
        dimension_semantics=("parallel",), vmem_limit_bytes=vmem_limit)

    tok_spec = pl.BlockSpec((1, 1, TBLK), lambda i: (i, 0, 0))
    table_spec = pl.BlockSpec((Vpad, Vpad), lambda i: (0, 0))
    logits_spec = pl.BlockSpec((TBLK, Vpad), lambda i: (i, 0))

    if has_targets:
        m_mat, lse_row = pl.pallas_call(
            _lse_m_kernel,
            out_shape=(
                jax.ShapeDtypeStruct((Vpad, Vpad), jnp.float32),
                jax.ShapeDtypeStruct((1, Vpad), jnp.float32),
            ),
        )(table_pad)

        tgt_flat = targets.reshape(-1).astype(jnp.int32)
        if Npad > N:
            tgt_flat = jnp.pad(tgt_flat, (0, Npad - N))
        tgt3 = tgt_flat.reshape(num_tiles, 1, TBLK)

        logits_pad, partials = pl.pallas_call(
            _loss_tile_kernel,
            out_shape=(
                jax.ShapeDtypeStruct((Npad, Vpad), jnp.float32),
                jax.ShapeDtypeStruct((num_tiles, 1, Vpad), jnp.float32),
            ),
            grid_spec=pltpu.PrefetchScalarGridSpec(
                num_scalar_prefetch=0,
                grid=(num_tiles,),
                in_specs=[tok_spec, tok_spec, table_spec,
                          pl.BlockSpec((Vpad, Vpad), lambda i: (0, 0))],
                out_specs=[logits_spec,
                           pl.BlockSpec((1, 1, Vpad), lambda i: (i, 0, 0))],
            ),
            compiler_params=cparams,
        )(idx3, tgt3, table_bf16, m_mat)

        loss_sum = jnp.sum(partials)
        if Npad > N:
            loss_sum = loss_sum - jnp.float32(Npad - N) * m_mat[0, 0]
        loss = loss_sum / jnp.float32(N)
        logits_flat = logits_pad[:N, :V] if (Npad > N or Vpad > V) \
            else logits_pad
        return logits_flat, loss

    logits_pad = pl.pallas_call(
        _logits_tile_kernel,
        out_shape=jax.ShapeDtypeStruct((Npad, Vpad), jnp.float32),
        grid_spec=pltpu.PrefetchScalarGridSpec(
            num_scalar_prefetch=0,
            grid=(num_tiles,),
            in_specs=[tok_spec, table_spec],
            out_specs=logits_spec,
        ),
        compiler_params=cparams,
    )(idx3, table_bf16)
    if Npad > N or Vpad > V:
        logits_pad = logits_pad[:N, :V]
    return logits_pad.reshape(B, T, V), None


def kernel(idx, targets, table):
    return _forward(idx, targets, table)


# speedup vs baseline: 6.9926x; 1.5067x over previous
import functools

import jax
import jax.numpy as jnp
from jax.experimental import pallas as pl
from jax.experimental.pallas import tpu as pltpu


def _round_up(x, m):
    return (x + m - 1) // m * m


def _lse_m_kernel(table_ref, m_ref, lse_ref):
    t = table_ref[...]
    mx = jnp.max(t, axis=1, keepdims=True)
    lse = jnp.log(jnp.sum(jnp.exp(t - mx), axis=1, keepdims=True)) + mx
    m_ref[...] = lse - t
    lse_ref[...] = lse.reshape(1, -1)


def _loss_tile_kernel(idx_ref, tgt_ref, table_ref, m_ref,
                      logits_ref, partial_ref):
    tblk, vpad = logits_ref.shape
    tok = idx_ref[...].reshape(1, tblk)
    tgt = tgt_ref[...].reshape(1, tblk)

    viota = jax.lax.broadcasted_iota(jnp.int32, (vpad, tblk), 0)
    p = jnp.where(tok == viota, 1.0, 0.0).astype(jnp.bfloat16)
    q = jnp.where(tgt == viota, 1.0, 0.0).astype(jnp.bfloat16)

    logits_ref[...] = jax.lax.dot_general(
        p, table_ref[...], (((0,), (0,)), ((), ())),
        preferred_element_type=jnp.float32)

    c = jax.lax.dot_general(
        p, q, (((1,), (1,)), ((), ())),
        preferred_element_type=jnp.float32)
    partial = jnp.sum(c * m_ref[...], axis=0, keepdims=True)
    partial_ref[...] = partial.reshape(1, 1, vpad)


def _logits_tile_kernel(idx_ref, table_ref, logits_ref):
    tblk, vpad = logits_ref.shape
    tok = idx_ref[...].reshape(1, tblk)
    viota = jax.lax.broadcasted_iota(jnp.int32, (vpad, tblk), 0)
    p = jnp.where(tok == viota, 1.0, 0.0).astype(jnp.bfloat16)
    logits_ref[...] = jax.lax.dot_general(
        p, table_ref[...], (((0,), (0,)), ((), ())),
        preferred_element_type=jnp.float32)


@functools.partial(jax.jit, static_argnames=("tblk",))
def _forward(idx, targets, table, *, tblk=1024):
    B, T = idx.shape
    V = table.shape[0]
    N = B * T

    Vpad = _round_up(V, 128)
    TBLK = min(tblk, _round_up(N, 128))
    Npad = _round_up(N, TBLK)
    num_tiles = Npad // TBLK
    has_targets = targets is not None

    table_f32 = table.astype(jnp.float32)
    table_pad = jnp.pad(table_f32, ((0, Vpad - V), (0, Vpad - V)))
    if has_targets and Vpad > V:
        table_pad = table_pad.at[:, V:].set(jnp.float32(-1e30))
    table_bf16 = table_pad.astype(jnp.bfloat16)

    idx_flat = idx.reshape(-1).astype(jnp.int32)
    if Npad > N:
        idx_flat = jnp.pad(idx_flat, (0, Npad - N))
    idx3 = idx_flat.reshape(num_tiles, 1, TBLK)

    vmem_limit = int(min(96 * 1024 * 1024,
                         max(8 * 1024 * 1024,
                             8 * TBLK * Vpad * 4 + 6 * Vpad * Vpad * 4)))
    cparams = pltpu.CompilerParams(
        dimension_semantics=("parallel",), vmem_limit_bytes=vmem_limit)

    tok_spec = pl.BlockSpec((1, 1, TBLK), lambda i: (i, 0, 0))
    table_spec = pl.BlockSpec((Vpad, Vpad), lambda i: (0, 0))
    logits_spec = pl.BlockSpec((TBLK, Vpad), lambda i: (i, 0))

    if has_targets:
        m_mat, lse_row = pl.pallas_call(
            _lse_m_kernel,
            out_shape=(
                jax.ShapeDtypeStruct((Vpad, Vpad), jnp.float32),
                jax.ShapeDtypeStruct((1, Vpad), jnp.float32),
            ),
        )(table_pad)

        tgt_flat = targets.reshape(-1).astype(jnp.int32)
        if Npad > N:
            tgt_flat = jnp.pad(tgt_flat, (0, Npad - N))
        tgt3 = tgt_flat.reshape(num_tiles, 1, TBLK)

        logits_pad, partials = pl.pallas_call(
            _loss_tile_kernel,
            out_shape=(
                jax.ShapeDtypeStruct((Npad, Vpad), jnp.float32),
                jax.ShapeDtypeStruct((num_tiles, 1, Vpad), jnp.float32),
            ),
            grid_spec=pltpu.PrefetchScalarGridSpec(
                num_scalar_prefetch=0,
                grid=(num_tiles,),
                in_specs=[tok_spec, tok_spec, table_spec,
                          pl.BlockSpec((Vpad, Vpad), lambda i: (0, 0))],
                out_specs=[logits_spec,
                           pl.BlockSpec((1, 1, Vpad), lambda i: (i, 0, 0))],
            ),
            compiler_params=cparams,
        )(idx3, tgt3, table_bf16, m_mat)

        loss_sum = jnp.sum(partials)
        if Npad > N:
            loss_sum = loss_sum - jnp.float32(Npad - N) * m_mat[0, 0]
        loss = loss_sum / jnp.float32(N)
        logits_flat = logits_pad[:N, :V] if (Npad > N or Vpad > V) \
            else logits_pad
        return logits_flat, loss

    logits_pad = pl.pallas_call(
        _logits_tile_kernel,
        out_shape=jax.ShapeDtypeStruct((Npad, Vpad), jnp.float32),
        grid_spec=pltpu.PrefetchScalarGridSpec(
            num_scalar_prefetch=0,
            grid=(num_tiles,),
            in_specs=[tok_spec, table_spec],
            out_specs=logits_spec,
        ),
        compiler_params=cparams,
    )(idx3, table_bf16)
    if Npad > N or Vpad > V:
        logits_pad = logits_pad[:N, :V]
    return logits_pad.reshape(B, T, V), None


def kernel(idx, targets, table):
    return _forward(idx, targets, table)


# speedup vs baseline: 9.7191x; 1.3899x over previous
import functools

import jax
import jax.numpy as jnp
from jax.experimental import pallas as pl
from jax.experimental.pallas import tpu as pltpu


def _round_up(x, m):
    return (x + m - 1) // m * m


def _lse_m_kernel(table_ref, m_ref, lse_ref):
    t = table_ref[...]
    mx = jnp.max(t, axis=1, keepdims=True)
    lse = jnp.log(jnp.sum(jnp.exp(t - mx), axis=1, keepdims=True)) + mx
    m_ref[...] = lse - t
    lse_ref[...] = lse.reshape(1, -1)


def _loss_tile_kernel(idx_ref, tgt_ref, table_ref, m_ref,
                      logits_ref, partial_ref):
    tblk, vpad = logits_ref.shape
    tok = idx_ref[...].reshape(1, tblk)
    tgt = tgt_ref[...].reshape(1, tblk)

    viota = jax.lax.broadcasted_iota(jnp.int32, (vpad, tblk), 0)
    p = jnp.where(tok == viota, 1.0, 0.0).astype(jnp.bfloat16)
    q = jnp.where(tgt == viota, 1.0, 0.0).astype(jnp.bfloat16)

    logits_ref[...] = jax.lax.dot_general(
        p, table_ref[...], (((0,), (0,)), ((), ())),
        preferred_element_type=jnp.float32)

    c = jax.lax.dot_general(
        p, q, (((1,), (1,)), ((), ())),
        preferred_element_type=jnp.float32)
    partial = jnp.sum(c * m_ref[...], axis=0, keepdims=True)
    partial_ref[...] = partial.reshape(1, 1, vpad)


def _logits_tile_kernel(idx_ref, table_ref, logits_ref):
    tblk, vpad = logits_ref.shape
    tok = idx_ref[...].reshape(1, tblk)
    viota = jax.lax.broadcasted_iota(jnp.int32, (vpad, tblk), 0)
    p = jnp.where(tok == viota, 1.0, 0.0).astype(jnp.bfloat16)
    logits_ref[...] = jax.lax.dot_general(
        p, table_ref[...], (((0,), (0,)), ((), ())),
        preferred_element_type=jnp.float32)


@functools.partial(jax.jit, static_argnames=("tblk",))
def _forward(idx, targets, table, *, tblk=2048):
    B, T = idx.shape
    V = table.shape[0]
    N = B * T

    Vpad = _round_up(V, 128)
    TBLK = min(tblk, _round_up(N, 128))
    Npad = _round_up(N, TBLK)
    num_tiles = Npad // TBLK
    has_targets = targets is not None

    table_f32 = table.astype(jnp.float32)
    table_pad = jnp.pad(table_f32, ((0, Vpad - V), (0, Vpad - V)))
    if has_targets and Vpad > V:
        table_pad = table_pad.at[:, V:].set(jnp.float32(-1e30))
    table_bf16 = table_pad.astype(jnp.bfloat16)

    idx_flat = idx.reshape(-1).astype(jnp.int32)
    if Npad > N:
        idx_flat = jnp.pad(idx_flat, (0, Npad - N))
    idx3 = idx_flat.reshape(num_tiles, 1, TBLK)

    vmem_limit = int(min(96 * 1024 * 1024,
                         max(8 * 1024 * 1024,
                             8 * TBLK * Vpad * 4 + 6 * Vpad * Vpad * 4)))
    cparams = pltpu.CompilerParams(
        dimension_semantics=("parallel",), vmem_limit_bytes=vmem_limit)

    tok_spec = pl.BlockSpec((1, 1, TBLK), lambda i: (i, 0, 0))
    table_spec = pl.BlockSpec((Vpad, Vpad), lambda i: (0, 0))
    logits_spec = pl.BlockSpec((TBLK, Vpad), lambda i: (i, 0))

    if has_targets:
        m_mat, lse_row = pl.pallas_call(
            _lse_m_kernel,
            out_shape=(
                jax.ShapeDtypeStruct((Vpad, Vpad), jnp.float32),
                jax.ShapeDtypeStruct((1, Vpad), jnp.float32),
            ),
        )(table_pad)

        tgt_flat = targets.reshape(-1).astype(jnp.int32)
        if Npad > N:
            tgt_flat = jnp.pad(tgt_flat, (0, Npad - N))
        tgt3 = tgt_flat.reshape(num_tiles, 1, TBLK)

        logits_pad, partials = pl.pallas_call(
            _loss_tile_kernel,
            out_shape=(
                jax.ShapeDtypeStruct((Npad, Vpad), jnp.float32),
                jax.ShapeDtypeStruct((num_tiles, 1, Vpad), jnp.float32),
            ),
            grid_spec=pltpu.PrefetchScalarGridSpec(
                num_scalar_prefetch=0,
                grid=(num_tiles,),
                in_specs=[tok_spec, tok_spec, table_spec,
                          pl.BlockSpec((Vpad, Vpad), lambda i: (0, 0))],
                out_specs=[logits_spec,
                           pl.BlockSpec((1, 1, Vpad), lambda i: (i, 0, 0))],
            ),
            compiler_params=cparams,
        )(idx3, tgt3, table_bf16, m_mat)

        loss_sum = jnp.sum(partials)
        if Npad > N:
            loss_sum = loss_sum - jnp.float32(Npad - N) * m_mat[0, 0]
        loss = loss_sum / jnp.float32(N)
        logits_flat = logits_pad[:N, :V] if (Npad > N or Vpad > V) \
            else logits_pad
        return logits_flat, loss

    logits_pad = pl.pallas_call(
        _logits_tile_kernel,
        out_shape=jax.ShapeDtypeStruct((Npad, Vpad), jnp.float32),
        grid_spec=pltpu.PrefetchScalarGridSpec(
            num_scalar_prefetch=0,
            grid=(num_tiles,),
            in_specs=[tok_spec, table_spec],
            out_specs=logits_spec,
        ),
        compiler_params=cparams,
    )(idx3, table_bf16)
    if Npad > N or Vpad > V:
        logits_pad = logits_pad[:N, :V]
    return logits_pad.reshape(B, T, V), None


def kernel(idx, targets, table):
    return _forward(idx, targets, table)


# speedup vs baseline: 12.1190x; 1.2469x over previous
import functools

import jax
import jax.numpy as jnp
from jax.experimental import pallas as pl
from jax.experimental.pallas import tpu as pltpu


def _round_up(x, m):
    return (x + m - 1) // m * m


def _lse_m_kernel(table_ref, m_ref, lse_ref):
    t = table_ref[...]
    mx = jnp.max(t, axis=1, keepdims=True)
    lse = jnp.log(jnp.sum(jnp.exp(t - mx), axis=1, keepdims=True)) + mx
    m_ref[...] = lse - t
    lse_ref[...] = lse.reshape(1, -1)


def _loss_tile_kernel(idx_ref, tgt_ref, table_ref, m_ref,
                      logits_ref, partial_ref):
    tblk, vpad = logits_ref.shape
    tok = idx_ref[...].reshape(1, tblk)
    tgt = tgt_ref[...].reshape(1, tblk)

    viota = jax.lax.broadcasted_iota(jnp.int32, (vpad, tblk), 0)
    p = jnp.where(tok == viota, 1.0, 0.0).astype(jnp.bfloat16)
    q = jnp.where(tgt == viota, 1.0, 0.0).astype(jnp.bfloat16)

    logits_ref[...] = jax.lax.dot_general(
        p, table_ref[...], (((0,), (0,)), ((), ())),
        preferred_element_type=jnp.float32)

    c = jax.lax.dot_general(
        p, q, (((1,), (1,)), ((), ())),
        preferred_element_type=jnp.float32)
    partial = jnp.sum(c * m_ref[...], axis=0, keepdims=True)
    partial_ref[...] = partial.reshape(1, 1, vpad)


def _logits_tile_kernel(idx_ref, table_ref, logits_ref):
    tblk, vpad = logits_ref.shape
    tok = idx_ref[...].reshape(1, tblk)
    viota = jax.lax.broadcasted_iota(jnp.int32, (vpad, tblk), 0)
    p = jnp.where(tok == viota, 1.0, 0.0).astype(jnp.bfloat16)
    logits_ref[...] = jax.lax.dot_general(
        p, table_ref[...], (((0,), (0,)), ((), ())),
        preferred_element_type=jnp.float32)


@functools.partial(jax.jit, static_argnames=("tblk",))
def _forward(idx, targets, table, *, tblk=4096):
    B, T = idx.shape
    V = table.shape[0]
    N = B * T

    Vpad = _round_up(V, 128)
    TBLK = min(tblk, _round_up(N, 128))
    Npad = _round_up(N, TBLK)
    num_tiles = Npad // TBLK
    has_targets = targets is not None

    table_f32 = table.astype(jnp.float32)
    table_pad = jnp.pad(table_f32, ((0, Vpad - V), (0, Vpad - V)))
    if has_targets and Vpad > V:
        table_pad = table_pad.at[:, V:].set(jnp.float32(-1e30))
    table_bf16 = table_pad.astype(jnp.bfloat16)

    idx_flat = idx.reshape(-1).astype(jnp.int32)
    if Npad > N:
        idx_flat = jnp.pad(idx_flat, (0, Npad - N))
    idx3 = idx_flat.reshape(num_tiles, 1, TBLK)

    vmem_limit = int(min(96 * 1024 * 1024,
                         max(8 * 1024 * 1024,
                             8 * TBLK * Vpad * 4 + 6 * Vpad * Vpad * 4)))
    cparams = pltpu.CompilerParams(
        dimension_semantics=("parallel",), vmem_limit_bytes=vmem_limit)

    tok_spec = pl.BlockSpec((1, 1, TBLK), lambda i: (i, 0, 0))
    table_spec = pl.BlockSpec((Vpad, Vpad), lambda i: (0, 0))
    logits_spec = pl.BlockSpec((TBLK, Vpad), lambda i: (i, 0))

    if has_targets:
        m_mat, lse_row = pl.pallas_call(
            _lse_m_kernel,
            out_shape=(
                jax.ShapeDtypeStruct((Vpad, Vpad), jnp.float32),
                jax.ShapeDtypeStruct((1, Vpad), jnp.float32),
            ),
        )(table_pad)

        tgt_flat = targets.reshape(-1).astype(jnp.int32)
        if Npad > N:
            tgt_flat = jnp.pad(tgt_flat, (0, Npad - N))
        tgt3 = tgt_flat.reshape(num_tiles, 1, TBLK)

        logits_pad, partials = pl.pallas_call(
            _loss_tile_kernel,
            out_shape=(
                jax.ShapeDtypeStruct((Npad, Vpad), jnp.float32),
                jax.ShapeDtypeStruct((num_tiles, 1, Vpad), jnp.float32),
            ),
            grid_spec=pltpu.PrefetchScalarGridSpec(
                num_scalar_prefetch=0,
                grid=(num_tiles,),
                in_specs=[tok_spec, tok_spec, table_spec,
                          pl.BlockSpec((Vpad, Vpad), lambda i: (0, 0))],
                out_specs=[logits_spec,
                           pl.BlockSpec((1, 1, Vpad), lambda i: (i, 0, 0))],
            ),
            compiler_params=cparams,
        )(idx3, tgt3, table_bf16, m_mat)

        loss_sum = jnp.sum(partials)
        if Npad > N:
            loss_sum = loss_sum - jnp.float32(Npad - N) * m_mat[0, 0]
        loss = loss_sum / jnp.float32(N)
        logits_flat = logits_pad[:N, :V] if (Npad > N or Vpad > V) \
            else logits_pad
        return logits_flat, loss

    logits_pad = pl.pallas_call(
        _logits_tile_kernel,
        out_shape=jax.ShapeDtypeStruct((Npad, Vpad), jnp.float32),
        grid_spec=pltpu.PrefetchScalarGridSpec(
            num_scalar_prefetch=0,
            grid=(num_tiles,),
            in_specs=[tok_spec, table_spec],
            out_specs=logits_spec,
        ),
        compiler_params=cparams,
    )(idx3, table_bf16)
    if Npad > N or Vpad > V:
        logits_pad = logits_pad[:N, :V]
    return logits_pad.reshape(B, T, V), None


def kernel(idx, targets, table):
    return _forward(idx, targets, table)


# speedup vs baseline: 13.9881x; 1.1542x over previous
import functools

import jax
import jax.numpy as jnp
from jax.experimental import pallas as pl
from jax.experimental.pallas import tpu as pltpu


def _round_up(x, m):
    return (x + m - 1) // m * m


def _lse_m_kernel(table_ref, m_ref, lse_ref):
    t = table_ref[...]
    mx = jnp.max(t, axis=1, keepdims=True)
    lse = jnp.log(jnp.sum(jnp.exp(t - mx), axis=1, keepdims=True)) + mx
    m_ref[...] = lse - t
    lse_ref[...] = lse.reshape(1, -1)


def _loss_tile_kernel(idx_ref, tgt_ref, table_ref, m_ref,
                      logits_ref, partial_ref):
    tblk, vpad = logits_ref.shape
    tok = idx_ref[...].reshape(1, tblk)
    tgt = tgt_ref[...].reshape(1, tblk)

    viota = jax.lax.broadcasted_iota(jnp.int32, (vpad, tblk), 0)
    p = jnp.where(tok == viota, 1.0, 0.0).astype(jnp.bfloat16)
    q = jnp.where(tgt == viota, 1.0, 0.0).astype(jnp.bfloat16)

    logits_ref[...] = jax.lax.dot_general(
        p, table_ref[...], (((0,), (0,)), ((), ())),
        preferred_element_type=jnp.float32)

    c = jax.lax.dot_general(
        p, q, (((1,), (1,)), ((), ())),
        preferred_element_type=jnp.float32)
    partial = jnp.sum(c * m_ref[...], axis=0, keepdims=True)
    partial_ref[...] = partial.reshape(1, 1, vpad)


def _logits_tile_kernel(idx_ref, table_ref, logits_ref):
    tblk, vpad = logits_ref.shape
    tok = idx_ref[...].reshape(1, tblk)
    viota = jax.lax.broadcasted_iota(jnp.int32, (vpad, tblk), 0)
    p = jnp.where(tok == viota, 1.0, 0.0).astype(jnp.bfloat16)
    logits_ref[...] = jax.lax.dot_general(
        p, table_ref[...], (((0,), (0,)), ((), ())),
        preferred_element_type=jnp.float32)


@functools.partial(jax.jit, static_argnames=("tblk",))
def _forward(idx, targets, table, *, tblk=8192):
    B, T = idx.shape
    V = table.shape[0]
    N = B * T

    Vpad = _round_up(V, 128)
    TBLK = min(tblk, _round_up(N, 128))
    Npad = _round_up(N, TBLK)
    num_tiles = Npad // TBLK
    has_targets = targets is not None

    table_f32 = table.astype(jnp.float32)
    table_pad = jnp.pad(table_f32, ((0, Vpad - V), (0, Vpad - V)))
    if has_targets and Vpad > V:
        table_pad = table_pad.at[:, V:].set(jnp.float32(-1e30))
    table_bf16 = table_pad.astype(jnp.bfloat16)

    idx_flat = idx.reshape(-1).astype(jnp.int32)
    if Npad > N:
        idx_flat = jnp.pad(idx_flat, (0, Npad - N))
    idx3 = idx_flat.reshape(num_tiles, 1, TBLK)

    vmem_limit = int(min(96 * 1024 * 1024,
                         max(8 * 1024 * 1024,
                             8 * TBLK * Vpad * 4 + 6 * Vpad * Vpad * 4)))
    cparams = pltpu.CompilerParams(
        dimension_semantics=("parallel",), vmem_limit_bytes=vmem_limit)

    tok_spec = pl.BlockSpec((1, 1, TBLK), lambda i: (i, 0, 0))
    table_spec = pl.BlockSpec((Vpad, Vpad), lambda i: (0, 0))
    logits_spec = pl.BlockSpec((TBLK, Vpad), lambda i: (i, 0))

    if has_targets:
        m_mat, lse_row = pl.pallas_call(
            _lse_m_kernel,
            out_shape=(
                jax.ShapeDtypeStruct((Vpad, Vpad), jnp.float32),
                jax.ShapeDtypeStruct((1, Vpad), jnp.float32),
            ),
        )(table_pad)

        tgt_flat = targets.reshape(-1).astype(jnp.int32)
        if Npad > N:
            tgt_flat = jnp.pad(tgt_flat, (0, Npad - N))
        tgt3 = tgt_flat.reshape(num_tiles, 1, TBLK)

        logits_pad, partials = pl.pallas_call(
            _loss_tile_kernel,
            out_shape=(
                jax.ShapeDtypeStruct((Npad, Vpad), jnp.float32),
                jax.ShapeDtypeStruct((num_tiles, 1, Vpad), jnp.float32),
            ),
            grid_spec=pltpu.PrefetchScalarGridSpec(
                num_scalar_prefetch=0,
                grid=(num_tiles,),
                in_specs=[tok_spec, tok_spec, table_spec,
                          pl.BlockSpec((Vpad, Vpad), lambda i: (0, 0))],
                out_specs=[logits_spec,
                           pl.BlockSpec((1, 1, Vpad), lambda i: (i, 0, 0))],
            ),
            compiler_params=cparams,
        )(idx3, tgt3, table_bf16, m_mat)

        loss_sum = jnp.sum(partials)
        if Npad > N:
            loss_sum = loss_sum - jnp.float32(Npad - N) * m_mat[0, 0]
        loss = loss_sum / jnp.float32(N)
        logits_flat = logits_pad[:N, :V] if (Npad > N or Vpad > V) \
            else logits_pad
        return logits_flat, loss

    logits_pad = pl.pallas_call(
        _logits_tile_kernel,
        out_shape=jax.ShapeDtypeStruct((Npad, Vpad), jnp.float32),
        grid_spec=pltpu.PrefetchScalarGridSpec(
            num_scalar_prefetch=0,
            grid=(num_tiles,),
            in_specs=[tok_spec, table_spec],
            out_specs=logits_spec,
        ),
        compiler_params=cparams,
    )(idx3, table_bf16)
    if Npad > N or Vpad > V:
        logits_pad = logits_pad[:N, :V]
    return logits_pad.reshape(B, T, V), None


def kernel(idx, targets, table):
    return _forward(idx, targets, table)


# speedup vs baseline: 14.9152x; 1.0663x over previous
import functools

import jax
import jax.numpy as jnp
from jax.experimental import pallas as pl
from jax.experimental.pallas import tpu as pltpu


def _round_up(x, m):
    return (x + m - 1) // m * m


def _lse_m_kernel(table_ref, m_ref, lse_ref):
    t = table_ref[...]
    mx = jnp.max(t, axis=1, keepdims=True)
    lse = jnp.log(jnp.sum(jnp.exp(t - mx), axis=1, keepdims=True)) + mx
    m_ref[...] = lse - t
    lse_ref[...] = lse.reshape(1, -1)


def _loss_tile_kernel(idx_ref, tgt_ref, table_ref, m_ref,
                      logits_ref, partial_ref):
    tblk, vpad = logits_ref.shape
    tok = idx_ref[...].reshape(1, tblk)
    tgt = tgt_ref[...].reshape(1, tblk)

    viota = jax.lax.broadcasted_iota(jnp.int32, (vpad, tblk), 0)
    p = jnp.where(tok == viota, 1.0, 0.0).astype(jnp.bfloat16)
    q = jnp.where(tgt == viota, 1.0, 0.0).astype(jnp.bfloat16)

    logits_ref[...] = jax.lax.dot_general(
        p, table_ref[...], (((0,), (0,)), ((), ())),
        preferred_element_type=jnp.float32)

    c = jax.lax.dot_general(
        p, q, (((1,), (1,)), ((), ())),
        preferred_element_type=jnp.float32)
    partial = jnp.sum(c * m_ref[...], axis=0, keepdims=True)
    partial_ref[...] = partial.reshape(1, 1, vpad)


def _logits_tile_kernel(idx_ref, table_ref, logits_ref):
    tblk, vpad = logits_ref.shape
    tok = idx_ref[...].reshape(1, tblk)
    viota = jax.lax.broadcasted_iota(jnp.int32, (vpad, tblk), 0)
    p = jnp.where(tok == viota, 1.0, 0.0).astype(jnp.bfloat16)
    logits_ref[...] = jax.lax.dot_general(
        p, table_ref[...], (((0,), (0,)), ((), ())),
        preferred_element_type=jnp.float32)


@functools.partial(jax.jit, static_argnames=("tblk",))
def _forward(idx, targets, table, *, tblk=16384):
    B, T = idx.shape
    V = table.shape[0]
    N = B * T

    Vpad = _round_up(V, 128)
    TBLK = min(tblk, _round_up(N, 128))
    Npad = _round_up(N, TBLK)
    num_tiles = Npad // TBLK
    has_targets = targets is not None

    table_f32 = table.astype(jnp.float32)
    table_pad = jnp.pad(table_f32, ((0, Vpad - V), (0, Vpad - V)))
    if has_targets and Vpad > V:
        table_pad = table_pad.at[:, V:].set(jnp.float32(-1e30))
    table_bf16 = table_pad.astype(jnp.bfloat16)

    idx_flat = idx.reshape(-1).astype(jnp.int32)
    if Npad > N:
        idx_flat = jnp.pad(idx_flat, (0, Npad - N))
    idx3 = idx_flat.reshape(num_tiles, 1, TBLK)

    vmem_limit = int(min(96 * 1024 * 1024,
                         max(8 * 1024 * 1024,
                             8 * TBLK * Vpad * 4 + 6 * Vpad * Vpad * 4)))
    cparams = pltpu.CompilerParams(
        dimension_semantics=("parallel",), vmem_limit_bytes=vmem_limit)

    tok_spec = pl.BlockSpec((1, 1, TBLK), lambda i: (i, 0, 0))
    table_spec = pl.BlockSpec((Vpad, Vpad), lambda i: (0, 0))
    logits_spec = pl.BlockSpec((TBLK, Vpad), lambda i: (i, 0))

    if has_targets:
        m_mat, lse_row = pl.pallas_call(
            _lse_m_kernel,
            out_shape=(
                jax.ShapeDtypeStruct((Vpad, Vpad), jnp.float32),
                jax.ShapeDtypeStruct((1, Vpad), jnp.float32),
            ),
        )(table_pad)

        tgt_flat = targets.reshape(-1).astype(jnp.int32)
        if Npad > N:
            tgt_flat = jnp.pad(tgt_flat, (0, Npad - N))
        tgt3 = tgt_flat.reshape(num_tiles, 1, TBLK)

        logits_pad, partials = pl.pallas_call(
            _loss_tile_kernel,
            out_shape=(
                jax.ShapeDtypeStruct((Npad, Vpad), jnp.float32),
                jax.ShapeDtypeStruct((num_tiles, 1, Vpad), jnp.float32),
            ),
            grid_spec=pltpu.PrefetchScalarGridSpec(
                num_scalar_prefetch=0,
                grid=(num_tiles,),
                in_specs=[tok_spec, tok_spec, table_spec,
                          pl.BlockSpec((Vpad, Vpad), lambda i: (0, 0))],
                out_specs=[logits_spec,
                           pl.BlockSpec((1, 1, Vpad), lambda i: (i, 0, 0))],
            ),
            compiler_params=cparams,
        )(idx3, tgt3, table_bf16, m_mat)

        loss_sum = jnp.sum(partials)
        if Npad > N:
            loss_sum = loss_sum - jnp.float32(Npad - N) * m_mat[0, 0]
        loss = loss_sum / jnp.float32(N)
        logits_flat = logits_pad[:N, :V] if (Npad > N or Vpad > V) \
            else logits_pad
        return logits_flat, loss

    logits_pad = pl.pallas_call(
        _logits_tile_kernel,
        out_shape=jax.ShapeDtypeStruct((Npad, Vpad), jnp.float32),
        grid_spec=pltpu.PrefetchScalarGridSpec(
            num_scalar_prefetch=0,
            grid=(num_tiles,),
            in_specs=[tok_spec, table_spec],
            out_specs=logits_spec,
        ),
        compiler_params=cparams,
    )(idx3, table_bf16)
    if Npad > N or Vpad > V:
        logits_pad = logits_pad[:N, :V]
    return logits_pad.reshape(B, T, V), None


def kernel(idx, targets, table):
    return _forward(idx, targets, table)
